# Initial kernel scaffold; baseline (speedup 1.0000x reference)
import jax
import jax.numpy as jnp
from jax.experimental import pallas as pl


def kernel(h, coord, edge_index, coord_diff, coord_cross, edge_attr, W1, b1, W2, b2, W3):
    raise NotImplementedError("write your pallas kernel here")


# speedup vs baseline: 3.9020x; 3.9020x over previous
import functools

import jax
import jax.numpy as jnp
from jax import lax
from jax.experimental import pallas as pl
from jax.experimental.pallas import tpu as pltpu
from jax.experimental.pallas import tpu_sc as plsc

N = 10000
E = 320000
H = 128

NC = 2
NS = 16
NW = NC * NS
EPW = E // NW

GCH = 80
NGCH = EPW // GCH

SCH = 2000
NSCH = EPW // SCH

RB = 2000
BE = 3200


def _tables_body(h_ref, w1a_ref, w1b_ref, b1_ref, a_ref, b_ref):
    hrows = h_ref[:]
    a_ref[:] = jnp.dot(hrows, w1a_ref[:], preferred_element_type=jnp.float32)
    b_ref[:] = jnp.dot(hrows, w1b_ref[:], preferred_element_type=jnp.float32) + b1_ref[:]


def _tables(h, w1a, w1b, b1r):
    return pl.pallas_call(
        _tables_body,
        grid=(N // RB,),
        in_specs=[
            pl.BlockSpec((RB, H), lambda i: (i, 0)),
            pl.BlockSpec((H, H), lambda i: (0, 0)),
            pl.BlockSpec((H, H), lambda i: (0, 0)),
            pl.BlockSpec((1, H), lambda i: (0, 0)),
        ],
        out_specs=[
            pl.BlockSpec((RB, H), lambda i: (i, 0)),
            pl.BlockSpec((RB, H), lambda i: (i, 0)),
        ],
        out_shape=[
            jax.ShapeDtypeStruct((N, H), jnp.float32),
            jax.ShapeDtypeStruct((N, H), jnp.float32),
        ],
    )(h, w1a, w1b, b1r)


def _gather_body(a_hbm, b_hbm, row_hbm, col_hbm, g1_hbm, g2_hbm,
                 idx_a, idx_b, buf_a, buf_b, sem_a, sem_b):
    wid = lax.axis_index("s") * NC + lax.axis_index("c")
    base = wid * EPW

    def chunk(i, carry):
        off = pl.multiple_of(base + i * GCH, 8)
        pltpu.sync_copy(row_hbm.at[pl.ds(off, GCH)], idx_a)
        pltpu.sync_copy(col_hbm.at[pl.ds(off, GCH)], idx_b)
        ca = pltpu.async_copy(a_hbm.at[idx_a], buf_a, sem_a)
        cb = pltpu.async_copy(b_hbm.at[idx_b], buf_b, sem_b)
        ca.wait()
        cb.wait()
        pltpu.sync_copy(buf_a, g1_hbm.at[pl.ds(off, GCH)])
        pltpu.sync_copy(buf_b, g2_hbm.at[pl.ds(off, GCH)])
        return carry

    lax.fori_loop(0, NGCH, chunk, 0)


def _gather(a_tab, b_tab, row, col):
    mesh = plsc.VectorSubcoreMesh(core_axis_name="c", subcore_axis_name="s")
    return pl.kernel(
        _gather_body,
        out_type=(
            jax.ShapeDtypeStruct((E, H), jnp.float32),
            jax.ShapeDtypeStruct((E, H), jnp.float32),
        ),
        mesh=mesh,
        compiler_params=pltpu.CompilerParams(needs_layout_passes=False),
        scratch_types=[
            pltpu.VMEM((GCH,), jnp.int32),
            pltpu.VMEM((GCH,), jnp.int32),
            pltpu.VMEM((GCH, H), jnp.float32),
            pltpu.VMEM((GCH, H), jnp.float32),
            pltpu.SemaphoreType.DMA,
            pltpu.SemaphoreType.DMA,
        ],
    )(a_tab, b_tab, row, col)


def _mlp_body(g1_ref, g2_ref, ea_ref, w1c_ref, w2_ref, b2_ref, w3_ref, phi_ref):
    x = g1_ref[:] + g2_ref[:] + ea_ref[:] * w1c_ref[:]
    x = jax.nn.silu(x)
    x = jax.nn.silu(jnp.dot(x, w2_ref[:], preferred_element_type=jnp.float32) + b2_ref[:])
    phi_ref[:] = jnp.dot(x, w3_ref[:], preferred_element_type=jnp.float32)


def _mlp(g1, g2, ea, w1c, w2, b2r, w3):
    return pl.pallas_call(
        _mlp_body,
        grid=(E // BE,),
        in_specs=[
            pl.BlockSpec((BE, H), lambda i: (i, 0)),
            pl.BlockSpec((BE, H), lambda i: (i, 0)),
            pl.BlockSpec((BE, 1), lambda i: (i, 0)),
            pl.BlockSpec((1, H), lambda i: (0, 0)),
            pl.BlockSpec((H, H), lambda i: (0, 0)),
            pl.BlockSpec((1, H), lambda i: (0, 0)),
            pl.BlockSpec((H, 1), lambda i: (0, 0)),
        ],
        out_specs=pl.BlockSpec((BE, 1), lambda i: (i, 0)),
        out_shape=jax.ShapeDtypeStruct((E, 1), jnp.float32),
    )(g1, g2, ea, w1c, w2, b2r, w3)


def _scatter_body(phi_hbm, cdt_hbm, row_hbm, part_hbm,
                  acc, idx_v, phi_v, cdx_v, cdy_v, cdz_v):
    wid = lax.axis_index("s") * NC + lax.axis_index("c")
    base = wid * EPW

    zeros16 = jnp.zeros((16,), jnp.float32)

    def zbody(i, carry):
        acc[pl.ds(i * 16, 16)] = zeros16
        return carry

    lax.fori_loop(0, (3 * N) // 16, zbody, 0)

    def chunk(i, carry):
        off = pl.multiple_of(base + i * SCH, 8)
        pltpu.sync_copy(row_hbm.at[pl.ds(off, SCH)], idx_v)
        pltpu.sync_copy(phi_hbm.at[pl.ds(off, SCH)], phi_v)
        pltpu.sync_copy(cdt_hbm.at[pl.ds(off, SCH)], cdx_v)
        pltpu.sync_copy(cdt_hbm.at[pl.ds(E + off, SCH)], cdy_v)
        pltpu.sync_copy(cdt_hbm.at[pl.ds(2 * E + off, SCH)], cdz_v)

        def group(j, c2):
            sl = pl.ds(j * 16, 16)
            ii = idx_v[sl]
            p = phi_v[sl]
            plsc.addupdate_scatter(acc, [ii], cdx_v[sl] * p)
            plsc.addupdate_scatter(acc, [ii + N], cdy_v[sl] * p)
            plsc.addupdate_scatter(acc, [ii + 2 * N], cdz_v[sl] * p)
            return c2

        lax.fori_loop(0, SCH // 16, group, 0)
        return carry

    lax.fori_loop(0, NSCH, chunk, 0)
    pltpu.sync_copy(acc, part_hbm.at[wid])


def _scatter(phi_flat, cdt_flat, row):
    mesh = plsc.VectorSubcoreMesh(core_axis_name="c", subcore_axis_name="s")
    return pl.kernel(
        _scatter_body,
        out_type=jax.ShapeDtypeStruct((NW, 3 * N), jnp.float32),
        mesh=mesh,
        compiler_params=pltpu.CompilerParams(needs_layout_passes=False),
        scratch_types=[
            pltpu.VMEM((3 * N,), jnp.float32),
            pltpu.VMEM((SCH,), jnp.int32),
            pltpu.VMEM((SCH,), jnp.float32),
            pltpu.VMEM((SCH,), jnp.float32),
            pltpu.VMEM((SCH,), jnp.float32),
            pltpu.VMEM((SCH,), jnp.float32),
        ],
    )(phi_flat, cdt_flat, row)


def _combine_body(part_ref, coordt_ref, out_ref):
    agg = jnp.sum(part_ref[:], axis=0, keepdims=True)
    out_ref[:] = coordt_ref[:] + agg * (1.0 / 100.0)


def _combine(partials, coordt):
    return pl.pallas_call(
        _combine_body,
        in_specs=[
            pl.BlockSpec((NW, 3 * N), lambda: (0, 0)),
            pl.BlockSpec((1, 3 * N), lambda: (0, 0)),
        ],
        out_specs=pl.BlockSpec((1, 3 * N), lambda: (0, 0)),
        out_shape=jax.ShapeDtypeStruct((1, 3 * N), jnp.float32),
    )(partials, coordt)


def kernel(h, coord, edge_index, coord_diff, coord_cross, edge_attr, W1, b1, W2, b2, W3):
    row = edge_index[0]
    col = edge_index[1]
    w1a = W1[:H]
    w1b = W1[H:2 * H]
    w1c = W1[2 * H:2 * H + 1]
    b1r = b1.reshape(1, H)
    b2r = b2.reshape(1, H)

    a_tab, b_tab = _tables(h, w1a, w1b, b1r)
    g1, g2 = _gather(a_tab, b_tab, row, col)
    phi = _mlp(g1, g2, edge_attr, w1c, W2, b2r, W3)

    cdt_flat = coord_diff.T.reshape(3 * E)
    partials = _scatter(phi.reshape(E), cdt_flat, row)

    coordt = coord.T.reshape(1, 3 * N)
    out_flat = _combine(partials, coordt)
    return out_flat.reshape(3, N).T
